# baseline (speedup 1.0000x reference)
import jax
import jax.numpy as jnp
from jax import lax
from jax.experimental import layout as jex_layout
from jax.experimental import pallas as pl
from jax.experimental.pallas import tpu as pltpu
from jax.experimental.pallas import tpu_sc as plsc

_INFO = plsc.get_sparse_core_info()
_NC = _INFO.num_cores
_NS = _INFO.num_subcores
_NW = _NC * _NS

_BATCH = 4096
_HIST = 200
_DIM = 64
_B = _BATCH * _HIST
_B_PER_W = _B // _NW
_BATCH_PER_W = _BATCH // _NW


def _gather_body(x_hbm, table_hbm, out_hbm, idx_v, rows_a, rows_b, sem_a, sem_b):
    wid = lax.axis_index("s") * _NC + lax.axis_index("c")
    base = wid * _B_PER_W
    bbase = wid * _BATCH_PER_W
    pltpu.sync_copy(x_hbm.at[pl.ds(base, _B_PER_W)], idx_v)

    def gather(k, buf, sem):
        kk = jnp.minimum(k, _BATCH_PER_W - 1)
        return pltpu.async_copy(
            table_hbm.at[idx_v.at[pl.ds(kk * _HIST, _HIST)]], buf, sem
        )

    gather(0, rows_a, sem_a)

    def step(i, _):
        k = 2 * i
        gather(k + 1, rows_b, sem_b)
        pltpu.make_async_copy(
            table_hbm.at[idx_v.at[pl.ds(0, _HIST)]], rows_a, sem_a
        ).wait()
        pltpu.sync_copy(rows_a, out_hbm.at[bbase + k])
        gather(k + 2, rows_a, sem_a)
        pltpu.make_async_copy(
            table_hbm.at[idx_v.at[pl.ds(0, _HIST)]], rows_b, sem_b
        ).wait()
        pltpu.sync_copy(rows_b, out_hbm.at[bbase + k + 1])
        return ()

    lax.fori_loop(0, _BATCH_PER_W // 2, step, (), unroll=False)
    pltpu.make_async_copy(
        table_hbm.at[idx_v.at[pl.ds(0, _HIST)]], rows_a, sem_a
    ).wait()


_gather = pl.kernel(
    _gather_body,
    mesh=plsc.VectorSubcoreMesh(core_axis_name="c", subcore_axis_name="s"),
    out_type=jax.ShapeDtypeStruct((_BATCH, _HIST, _DIM), jnp.float32),
    scratch_types=[
        pltpu.VMEM((_B_PER_W,), jnp.int32),
        pltpu.VMEM((_HIST, _DIM), jnp.float32),
        pltpu.VMEM((_HIST, _DIM), jnp.float32),
        pltpu.SemaphoreType.DMA,
        pltpu.SemaphoreType.DMA,
    ],
    compiler_params=pltpu.CompilerParams(use_tc_tiling_on_sc=False),
)


@jax.jit
def kernel(x, table):
    out = _gather(x.reshape(_B), table)
    return jex_layout.with_layout_constraint(
        out, jex_layout.Layout(major_to_minor=(0, 1, 2), tiling=())
    )


# speedup vs baseline: 1.2243x; 1.2243x over previous
import jax
import jax.numpy as jnp
from jax import lax
from jax.experimental import pallas as pl
from jax.experimental.pallas import tpu as pltpu
from jax.experimental.pallas import tpu_sc as plsc

_INFO = plsc.get_sparse_core_info()
_NC = _INFO.num_cores
_NS = _INFO.num_subcores
_NW = _NC * _NS

_BATCH = 4096
_HIST = 200
_DIM = 64
_PAD = 128
_B = _BATCH * _HIST
_B_PER_W = _B // _NW
_BATCH_PER_W = _BATCH // _NW


_CHUNK = 256
_NCHUNK = _B_PER_W // _CHUNK


def _gather_body(x_hbm, table_hbm, out_hbm, idx_v, rows_a, rows_b, sem_a, sem_b):
    wid = lax.axis_index("s") * _NC + lax.axis_index("c")
    base = wid * _B_PER_W
    pltpu.sync_copy(x_hbm.at[pl.ds(base, _B_PER_W)], idx_v)

    def gather(k, buf, sem):
        kk = jnp.minimum(k, _NCHUNK - 1)
        return pltpu.async_copy(
            table_hbm.at[idx_v.at[pl.ds(kk * _CHUNK, _CHUNK)]], buf, sem
        )

    def store(k, buf):
        pltpu.sync_copy(buf, out_hbm.at[pl.ds(base + k * _CHUNK, _CHUNK)])

    gather(0, rows_a, sem_a)

    def step(i, _):
        k = 2 * i
        gather(k + 1, rows_b, sem_b)
        pltpu.make_async_copy(
            table_hbm.at[idx_v.at[pl.ds(0, _CHUNK)]], rows_a, sem_a
        ).wait()
        store(k, rows_a)
        gather(k + 2, rows_a, sem_a)
        pltpu.make_async_copy(
            table_hbm.at[idx_v.at[pl.ds(0, _CHUNK)]], rows_b, sem_b
        ).wait()
        store(k + 1, rows_b)
        return ()

    lax.fori_loop(0, _NCHUNK // 2, step, (), unroll=False)
    pltpu.make_async_copy(
        table_hbm.at[idx_v.at[pl.ds(0, _CHUNK)]], rows_a, sem_a
    ).wait()


_gather = pl.kernel(
    _gather_body,
    mesh=plsc.VectorSubcoreMesh(core_axis_name="c", subcore_axis_name="s"),
    out_type=jax.ShapeDtypeStruct((_B, _PAD), jnp.float32),
    scratch_types=[
        pltpu.VMEM((_B_PER_W,), jnp.int32),
        pltpu.VMEM((_CHUNK, _PAD), jnp.float32),
        pltpu.VMEM((_CHUNK, _PAD), jnp.float32),
        pltpu.SemaphoreType.DMA,
        pltpu.SemaphoreType.DMA,
    ],
    compiler_params=pltpu.CompilerParams(use_tc_tiling_on_sc=True),
)


@jax.jit
def kernel(x, table):
    table_padded = jnp.pad(table, ((0, 0), (0, _PAD - _DIM)))
    res = _gather(x.reshape(_B), table_padded)
    return res[:, :_DIM].reshape(_BATCH, _HIST, _DIM)
